# Initial kernel scaffold; baseline (speedup 1.0000x reference)
import jax
import jax.numpy as jnp
from jax.experimental import pallas as pl


def kernel(g2m_efeat, grid_nfeat, mesh_nfeat, edge_index, edge_W1, edge_b1, edge_W2, edge_b2, edge_g, edge_bt, src_W1, src_b1, src_W2, src_b2, src_g, src_bt, dst_W1, dst_b1, dst_W2, dst_b2, dst_g, dst_bt):
    raise NotImplementedError("write your pallas kernel here")


# speedup vs baseline: 2.7741x; 2.7741x over previous
import functools

import jax
import jax.numpy as jnp
from jax import lax
from jax.experimental import pallas as pl
from jax.experimental.pallas import tpu as pltpu
from jax.experimental.pallas import tpu_sc as plsc

N_SRC = 100000
N_DST = 10000
E = 320000
D = 128
H = 128

NC = 2
NS = 16
NW = NC * NS
RW = E // NW
CH = 80
NCHUNK = RW // CH

f32 = jnp.float32

_sc_mesh = plsc.VectorSubcoreMesh(core_axis_name="c", subcore_axis_name="s")


def _ln(y, g, bt):
    mu = jnp.mean(y, axis=-1, keepdims=True)
    var = jnp.mean((y - mu) * (y - mu), axis=-1, keepdims=True)
    return (y - mu) * lax.rsqrt(var + 1e-5) * g + bt


def _silu(x):
    return x * lax.logistic(x)


def _grid_body(x_ref, wp_ref, w1_ref, b1_ref, w2_ref, b2_ref, g_ref, bt_ref,
               out_ref, proj_ref):
    x = x_ref[...]
    proj_ref[...] = jnp.dot(x, wp_ref[...], preferred_element_type=f32)
    h = _silu(jnp.dot(x, w1_ref[...], preferred_element_type=f32) + b1_ref[...])
    y = jnp.dot(h, w2_ref[...], preferred_element_type=f32) + b2_ref[...]
    out_ref[...] = x + _ln(y, g_ref[...], bt_ref[...])


def _meshproj_body(x_ref, wp_ref, proj_ref):
    proj_ref[...] = jnp.dot(x_ref[...], wp_ref[...], preferred_element_type=f32)


def _edge_body(ef_ref, gs_ref, gd_ref, w1_ref, b1_ref, w2_ref, b2_ref, g_ref,
               bt_ref, out_ref):
    pre = (jnp.dot(ef_ref[...], w1_ref[...], preferred_element_type=f32)
           + gs_ref[...] + gd_ref[...] + b1_ref[...])
    h = _silu(pre)
    y = jnp.dot(h, w2_ref[...], preferred_element_type=f32) + b2_ref[...]
    out_ref[...] = _ln(y, g_ref[...], bt_ref[...])


def _dst_body(p0_ref, p1_ref, m_ref, w1a_ref, w1b_ref, b1_ref, w2_ref, b2_ref,
              g_ref, bt_ref, out_ref):
    agg = p0_ref[...] + p1_ref[...]
    m = m_ref[...]
    pre = (jnp.dot(agg, w1a_ref[...], preferred_element_type=f32)
           + jnp.dot(m, w1b_ref[...], preferred_element_type=f32)
           + b1_ref[...])
    h = _silu(pre)
    y = jnp.dot(h, w2_ref[...], preferred_element_type=f32) + b2_ref[...]
    out_ref[...] = m + _ln(y, g_ref[...], bt_ref[...])


def _row_spec(br):
    return pl.BlockSpec((br, D), lambda i: (i, 0))


def _full_spec(shape):
    return pl.BlockSpec(shape, lambda i: tuple(0 for _ in shape))


def _sc_gather(grid_proj, mesh_proj, src_idx, dst_idx):
    @functools.partial(
        pl.kernel,
        out_type=(jax.ShapeDtypeStruct((E, D), f32),
                  jax.ShapeDtypeStruct((E, D), f32)),
        mesh=_sc_mesh,
        scratch_types=[
            pltpu.VMEM((CH,), jnp.int32),
            pltpu.VMEM((CH,), jnp.int32),
            pltpu.VMEM((CH, D), f32),
            pltpu.VMEM((CH, D), f32),
            pltpu.SemaphoreType.DMA,
            pltpu.SemaphoreType.DMA,
        ])
    def k(gp_hbm, mp_hbm, si_hbm, di_hbm, os_hbm, od_hbm,
          si_v, di_v, rs_v, rd_v, s1, s2):
        wid = lax.axis_index("s") * NC + lax.axis_index("c")
        base = wid * RW

        @pl.loop(0, NCHUNK)
        def _(c):
            off = base + c * CH
            pltpu.sync_copy(si_hbm.at[pl.ds(off, CH)], si_v)
            pltpu.sync_copy(di_hbm.at[pl.ds(off, CH)], di_v)
            c1 = pltpu.async_copy(gp_hbm.at[si_v], rs_v, s1)
            c2 = pltpu.async_copy(mp_hbm.at[di_v], rd_v, s2)
            c1.wait()
            c2.wait()
            pltpu.sync_copy(rs_v, os_hbm.at[pl.ds(off, CH)])
            pltpu.sync_copy(rd_v, od_hbm.at[pl.ds(off, CH)])

    return k(grid_proj, mesh_proj, src_idx, dst_idx)


def _sc_scatter(e_out, dst_idx, zeros_nd):
    @functools.partial(
        pl.kernel,
        out_type=jax.ShapeDtypeStruct((NC, N_DST, D), f32),
        mesh=_sc_mesh,
        scratch_types=[
            pltpu.VMEM((CH,), jnp.int32),
            pltpu.VMEM((CH, D), f32),
            pltpu.VMEM_SHARED((N_DST, D), f32),
        ])
    def k(eo_hbm, di_hbm, z_hbm, out_hbm, di_v, rows_v, agg_sh):
        cid = lax.axis_index("c")
        sid = lax.axis_index("s")
        wid = sid * NC + cid
        base = wid * RW

        @pl.when(sid == 0)
        def _():
            pltpu.sync_copy(z_hbm, agg_sh)
        plsc.subcore_barrier()

        @pl.loop(0, NCHUNK)
        def _(c):
            off = base + c * CH
            pltpu.sync_copy(di_hbm.at[pl.ds(off, CH)], di_v)
            pltpu.sync_copy(eo_hbm.at[pl.ds(off, CH)], rows_v)
            pltpu.sync_copy(rows_v, agg_sh.at[di_v], add=True)

        plsc.subcore_barrier()

        @pl.when(sid == 0)
        def _():
            pltpu.sync_copy(agg_sh, out_hbm.at[cid])

    return k(e_out, dst_idx, zeros_nd)


@jax.jit
def kernel(g2m_efeat, grid_nfeat, mesh_nfeat, edge_index,
           edge_W1, edge_b1, edge_W2, edge_b2, edge_g, edge_bt,
           src_W1, src_b1, src_W2, src_b2, src_g, src_bt,
           dst_W1, dst_b1, dst_W2, dst_b2, dst_g, dst_bt):
    src_idx = edge_index[0].astype(jnp.int32)
    dst_idx = edge_index[1].astype(jnp.int32)

    w1e = edge_W1[:D]
    w1s = edge_W1[D:2 * D]
    w1d = edge_W1[2 * D:]
    dw1a = dst_W1[:D]
    dw1b = dst_W1[D:]

    def r2(b):
        return b.reshape(1, -1)

    BRG = 2000
    grid_out, grid_proj = pl.pallas_call(
        _grid_body,
        grid=(N_SRC // BRG,),
        in_specs=[_row_spec(BRG), _full_spec((D, H)), _full_spec((D, H)),
                  _full_spec((1, H)), _full_spec((H, D)), _full_spec((1, D)),
                  _full_spec((1, D)), _full_spec((1, D))],
        out_specs=[_row_spec(BRG), _row_spec(BRG)],
        out_shape=[jax.ShapeDtypeStruct((N_SRC, D), f32),
                   jax.ShapeDtypeStruct((N_SRC, H), f32)],
    )(grid_nfeat, w1s, src_W1, r2(src_b1), src_W2, r2(src_b2), r2(src_g),
      r2(src_bt))

    BRM = 2000
    mesh_proj = pl.pallas_call(
        _meshproj_body,
        grid=(N_DST // BRM,),
        in_specs=[_row_spec(BRM), _full_spec((D, H))],
        out_specs=_row_spec(BRM),
        out_shape=jax.ShapeDtypeStruct((N_DST, H), f32),
    )(mesh_nfeat, w1d)

    g_src, g_dst = _sc_gather(grid_proj, mesh_proj, src_idx, dst_idx)

    BRE = 2000
    e_out = pl.pallas_call(
        _edge_body,
        grid=(E // BRE,),
        in_specs=[_row_spec(BRE), _row_spec(BRE), _row_spec(BRE),
                  _full_spec((D, H)), _full_spec((1, H)), _full_spec((H, D)),
                  _full_spec((1, D)), _full_spec((1, D)), _full_spec((1, D))],
        out_specs=_row_spec(BRE),
        out_shape=jax.ShapeDtypeStruct((E, D), f32),
    )(g2m_efeat, g_src, g_dst, w1e, r2(edge_b1), edge_W2, r2(edge_b2),
      r2(edge_g), r2(edge_bt))

    parts = _sc_scatter(e_out, dst_idx, jnp.zeros((N_DST, D), f32))

    BRD = 2000
    mesh_out = pl.pallas_call(
        _dst_body,
        grid=(N_DST // BRD,),
        in_specs=[_row_spec(BRD), _row_spec(BRD), _row_spec(BRD),
                  _full_spec((D, H)), _full_spec((D, H)), _full_spec((1, H)),
                  _full_spec((H, D)), _full_spec((1, D)), _full_spec((1, D)),
                  _full_spec((1, D))],
        out_specs=_row_spec(BRD),
        out_shape=jax.ShapeDtypeStruct((N_DST, D), f32),
    )(parts[0], parts[1], mesh_nfeat, dw1a, dw1b, r2(dst_b1), dst_W2,
      r2(dst_b2), r2(dst_g), r2(dst_bt))

    return (grid_out, mesh_out)


# speedup vs baseline: 4.2431x; 1.5296x over previous
import functools

import jax
import jax.numpy as jnp
from jax import lax
from jax.experimental import pallas as pl
from jax.experimental.pallas import tpu as pltpu
from jax.experimental.pallas import tpu_sc as plsc

N_SRC = 100000
N_DST = 10000
E = 320000
D = 128
H = 128

NC = 2
NS = 16
NW = NC * NS
RW = E // NW
CH = 80
NCH = RW // CH

f32 = jnp.float32

_sc_mesh = plsc.VectorSubcoreMesh(core_axis_name="c", subcore_axis_name="s")


def _ln(y, g, bt):
    mu = jnp.mean(y, axis=-1, keepdims=True)
    var = jnp.mean((y - mu) * (y - mu), axis=-1, keepdims=True)
    return (y - mu) * lax.rsqrt(var + 1e-5) * g + bt


def _silu(x):
    return x * lax.logistic(x)


def _grid_body(x_ref, wp_ref, w1_ref, b1_ref, w2_ref, b2_ref, g_ref, bt_ref,
               out_ref, proj_ref):
    x = x_ref[...]
    proj_ref[...] = jnp.dot(x, wp_ref[...], preferred_element_type=f32)
    h = _silu(jnp.dot(x, w1_ref[...], preferred_element_type=f32) + b1_ref[...])
    y = jnp.dot(h, w2_ref[...], preferred_element_type=f32) + b2_ref[...]
    out_ref[...] = x + _ln(y, g_ref[...], bt_ref[...])


def _meshproj_body(x_ref, wp_ref, proj_ref):
    proj_ref[...] = jnp.dot(x_ref[...], wp_ref[...], preferred_element_type=f32)


def _edge_body(ef_ref, gsum_ref, w1_ref, b1_ref, w2_ref, b2_ref, g_ref,
               bt_ref, out_ref):
    pre = (jnp.dot(ef_ref[...], w1_ref[...], preferred_element_type=f32)
           + gsum_ref[...] + b1_ref[...])
    h = _silu(pre)
    y = jnp.dot(h, w2_ref[...], preferred_element_type=f32) + b2_ref[...]
    out_ref[...] = _ln(y, g_ref[...], bt_ref[...])


def _dst_body(p0_ref, p1_ref, m_ref, w1a_ref, w1b_ref, b1_ref, w2_ref, b2_ref,
              g_ref, bt_ref, out_ref):
    agg = p0_ref[...] + p1_ref[...]
    m = m_ref[...]
    pre = (jnp.dot(agg, w1a_ref[...], preferred_element_type=f32)
           + jnp.dot(m, w1b_ref[...], preferred_element_type=f32)
           + b1_ref[...])
    h = _silu(pre)
    y = jnp.dot(h, w2_ref[...], preferred_element_type=f32) + b2_ref[...]
    out_ref[...] = m + _ln(y, g_ref[...], bt_ref[...])


def _row_spec(br):
    return pl.BlockSpec((br, D), lambda i: (i, 0))


def _full_spec(shape):
    return pl.BlockSpec(shape, lambda i: tuple(0 for _ in shape))


def _sc_gather(grid_proj, mesh_proj, si3, di3):
    @functools.partial(
        pl.kernel,
        out_type=jax.ShapeDtypeStruct((E, D), f32),
        mesh=_sc_mesh,
        scratch_types=[
            pltpu.VMEM((NCH, CH), jnp.int32),
            pltpu.VMEM((NCH, CH), jnp.int32),
            pltpu.VMEM((CH, D), f32), pltpu.VMEM((CH, D), f32),
            pltpu.VMEM((CH, D), f32), pltpu.VMEM((CH, D), f32),
            pltpu.VMEM((CH, D), f32), pltpu.VMEM((CH, D), f32),
            pltpu.SemaphoreType.DMA, pltpu.SemaphoreType.DMA,
            pltpu.SemaphoreType.DMA, pltpu.SemaphoreType.DMA,
            pltpu.SemaphoreType.DMA, pltpu.SemaphoreType.DMA,
        ])
    def k(gp_hbm, mp_hbm, si_hbm, di_hbm, out_hbm,
          si_v, di_v, rs0, rd0, rs1, rd1, rs2, rd2, g0, g1, g2, w0, w1, w2):
        rs = (rs0, rs1, rs2)
        rd = (rd0, rd1, rd2)
        gsem = (g0, g1, g2)
        wsem = (w0, w1, w2)
        wid = lax.axis_index("s") * NC + lax.axis_index("c")
        base = wid * RW
        pltpu.sync_copy(si_hbm.at[wid], si_v)
        pltpu.sync_copy(di_hbm.at[wid], di_v)

        def gstart(b, c):
            pltpu.async_copy(gp_hbm.at[si_v.at[c]], rs[b], gsem[b])
            pltpu.async_copy(mp_hbm.at[di_v.at[c]], rd[b], gsem[b])

        def gwait(b):
            pltpu.make_async_copy(gp_hbm.at[pl.ds(0, CH)], rs[b], gsem[b]).wait()
            pltpu.make_async_copy(gp_hbm.at[pl.ds(0, CH)], rd[b], gsem[b]).wait()

        def add_rows(b):
            @pl.loop(0, CH)
            def _(r):
                for co in range(D // 16):
                    sl = pl.ds(co * 16, 16)
                    rs[b][r, sl] = rs[b][r, sl] + rd[b][r, sl]

        def wstart(b, c):
            pltpu.async_copy(rs[b], out_hbm.at[pl.ds(base + c * CH, CH)],
                             wsem[b])

        def wwait(b):
            pltpu.make_async_copy(rs[b], out_hbm.at[pl.ds(0, CH)],
                                  wsem[b]).wait()

        gstart(0, 0)

        @pl.loop(0, (NCH - 2) // 3)
        def _(p):
            c = 3 * p
            for h in range(3):
                bh, bn = h, (h + 1) % 3
                if h < 2:
                    @pl.when(p > 0)
                    def _():
                        wwait(bn)
                else:
                    wwait(bn)
                gwait(bh)
                gstart(bn, c + h + 1)
                add_rows(bh)
                wstart(bh, c + h)

        wwait(1)
        gwait(0)
        gstart(1, NCH - 1)
        add_rows(0)
        wstart(0, NCH - 2)
        gwait(1)
        add_rows(1)
        wstart(1, NCH - 1)
        wwait(2)
        wwait(0)
        wwait(1)

    return k(grid_proj, mesh_proj, si3, di3)


def _sc_scatter(e_out, di3, zeros_nd):
    @functools.partial(
        pl.kernel,
        out_type=jax.ShapeDtypeStruct((NC, N_DST, D), f32),
        mesh=_sc_mesh,
        scratch_types=[
            pltpu.VMEM((NCH, CH), jnp.int32),
            pltpu.VMEM((CH, D), f32), pltpu.VMEM((CH, D), f32),
            pltpu.VMEM_SHARED((N_DST, D), f32),
            pltpu.SemaphoreType.DMA, pltpu.SemaphoreType.DMA,
        ])
    def k(eo_hbm, di_hbm, z_hbm, out_hbm, di_v, ra, rb, agg_sh, sa, sb):
        rows = (ra, rb)
        sem = (sa, sb)
        cid = lax.axis_index("c")
        sid = lax.axis_index("s")
        wid = sid * NC + cid
        base = wid * RW
        pltpu.sync_copy(di_hbm.at[wid], di_v)

        @pl.when(sid == 0)
        def _():
            pltpu.sync_copy(z_hbm, agg_sh)
        plsc.subcore_barrier()

        def rstart(b, c):
            pltpu.async_copy(eo_hbm.at[pl.ds(base + c * CH, CH)], rows[b],
                             sem[b])

        def rwait(b):
            pltpu.make_async_copy(eo_hbm.at[pl.ds(0, CH)], rows[b],
                                  sem[b]).wait()

        def scat(b, c):
            pltpu.sync_copy(rows[b], agg_sh.at[di_v.at[c]], add=True)

        rstart(0, 0)

        @pl.loop(0, (NCH - 1) // 2)
        def _(p):
            c = 2 * p
            rwait(0)
            rstart(1, c + 1)
            scat(0, c)
            rwait(1)
            @pl.when(p < (NCH - 1) // 2 - 1)
            def _():
                rstart(0, c + 2)
            scat(1, c + 1)

        rstart(0, NCH - 1)
        rwait(0)
        scat(0, NCH - 1)

        plsc.subcore_barrier()

        @pl.when(sid == 0)
        def _():
            pltpu.sync_copy(agg_sh, out_hbm.at[cid])

    return k(e_out, di3, zeros_nd)


@jax.jit
def kernel(g2m_efeat, grid_nfeat, mesh_nfeat, edge_index,
           edge_W1, edge_b1, edge_W2, edge_b2, edge_g, edge_bt,
           src_W1, src_b1, src_W2, src_b2, src_g, src_bt,
           dst_W1, dst_b1, dst_W2, dst_b2, dst_g, dst_bt):
    si3 = edge_index[0].astype(jnp.int32).reshape(NW, NCH, CH)
    di3 = edge_index[1].astype(jnp.int32).reshape(NW, NCH, CH)

    w1e = edge_W1[:D]
    w1s = edge_W1[D:2 * D]
    w1d = edge_W1[2 * D:]
    dw1a = dst_W1[:D]
    dw1b = dst_W1[D:]

    def r2(b):
        return b.reshape(1, -1)

    BRG = 2000
    grid_out, grid_proj = pl.pallas_call(
        _grid_body,
        grid=(N_SRC // BRG,),
        in_specs=[_row_spec(BRG), _full_spec((D, H)), _full_spec((D, H)),
                  _full_spec((1, H)), _full_spec((H, D)), _full_spec((1, D)),
                  _full_spec((1, D)), _full_spec((1, D))],
        out_specs=[_row_spec(BRG), _row_spec(BRG)],
        out_shape=[jax.ShapeDtypeStruct((N_SRC, D), f32),
                   jax.ShapeDtypeStruct((N_SRC, H), f32)],
    )(grid_nfeat, w1s, src_W1, r2(src_b1), src_W2, r2(src_b2), r2(src_g),
      r2(src_bt))

    BRM = 2000
    mesh_proj = pl.pallas_call(
        _meshproj_body,
        grid=(N_DST // BRM,),
        in_specs=[_row_spec(BRM), _full_spec((D, H))],
        out_specs=_row_spec(BRM),
        out_shape=jax.ShapeDtypeStruct((N_DST, H), f32),
    )(mesh_nfeat, w1d)

    gsum = _sc_gather(grid_proj, mesh_proj, si3, di3)

    BRE = 4000
    e_out = pl.pallas_call(
        _edge_body,
        grid=(E // BRE,),
        in_specs=[_row_spec(BRE), _row_spec(BRE),
                  _full_spec((D, H)), _full_spec((1, H)), _full_spec((H, D)),
                  _full_spec((1, D)), _full_spec((1, D)), _full_spec((1, D))],
        out_specs=_row_spec(BRE),
        out_shape=jax.ShapeDtypeStruct((E, D), f32),
    )(g2m_efeat, gsum, w1e, r2(edge_b1), edge_W2, r2(edge_b2),
      r2(edge_g), r2(edge_bt))

    parts = _sc_scatter(e_out, di3, jnp.zeros((N_DST, D), f32))

    BRD = 2000
    mesh_out = pl.pallas_call(
        _dst_body,
        grid=(N_DST // BRD,),
        in_specs=[_row_spec(BRD), _row_spec(BRD), _row_spec(BRD),
                  _full_spec((D, H)), _full_spec((D, H)), _full_spec((1, H)),
                  _full_spec((H, D)), _full_spec((1, D)), _full_spec((1, D)),
                  _full_spec((1, D))],
        out_specs=_row_spec(BRD),
        out_shape=jax.ShapeDtypeStruct((N_DST, D), f32),
    )(parts[0], parts[1], mesh_nfeat, dw1a, dw1b, r2(dst_b1), dst_W2,
      r2(dst_b2), r2(dst_g), r2(dst_bt))

    return (grid_out, mesh_out)


# speedup vs baseline: 4.4844x; 1.0569x over previous
import functools

import jax
import jax.numpy as jnp
from jax import lax
from jax.experimental import pallas as pl
from jax.experimental.pallas import tpu as pltpu
from jax.experimental.pallas import tpu_sc as plsc

N_SRC = 100000
N_DST = 10000
E = 320000
D = 128
H = 128

NC = 2
NS = 16
NW = NC * NS
RW = E // NW
CH = 80
NCH = RW // CH

f32 = jnp.float32

_sc_mesh = plsc.VectorSubcoreMesh(core_axis_name="c", subcore_axis_name="s")


def _ln(y, g, bt):
    mu = jnp.mean(y, axis=-1, keepdims=True)
    var = jnp.mean((y - mu) * (y - mu), axis=-1, keepdims=True)
    return (y - mu) * lax.rsqrt(var + 1e-5) * g + bt


def _silu(x):
    return x * lax.logistic(x)


def _grid_body(x_ref, w1_ref, b1_ref, w2_ref, b2_ref, g_ref, bt_ref,
               out_ref):
    x = x_ref[...]
    h = _silu(jnp.dot(x, w1_ref[...], preferred_element_type=f32) + b1_ref[...])
    y = jnp.dot(h, w2_ref[...], preferred_element_type=f32) + b2_ref[...]
    out_ref[...] = x + _ln(y, g_ref[...], bt_ref[...])


def _meshproj_body(x_ref, wp_ref, proj_ref):
    proj_ref[...] = jnp.dot(x_ref[...], wp_ref[...], preferred_element_type=f32)


def _edge_body(ef_ref, gsum_ref, w1_ref, b1_ref, w2_ref, b2_ref, g_ref,
               bt_ref, out_ref):
    pre = (jnp.dot(ef_ref[...], w1_ref[...], preferred_element_type=f32)
           + gsum_ref[...] + b1_ref[...])
    h = _silu(pre)
    y = jnp.dot(h, w2_ref[...], preferred_element_type=f32) + b2_ref[...]
    out_ref[...] = _ln(y, g_ref[...], bt_ref[...])


def _dst_body(p0_ref, p1_ref, m_ref, w1a_ref, w1b_ref, b1_ref, w2_ref, b2_ref,
              g_ref, bt_ref, out_ref):
    agg = p0_ref[...] + p1_ref[...]
    m = m_ref[...]
    pre = (jnp.dot(agg, w1a_ref[...], preferred_element_type=f32)
           + jnp.dot(m, w1b_ref[...], preferred_element_type=f32)
           + b1_ref[...])
    h = _silu(pre)
    y = jnp.dot(h, w2_ref[...], preferred_element_type=f32) + b2_ref[...]
    out_ref[...] = m + _ln(y, g_ref[...], bt_ref[...])


def _row_spec(br):
    return pl.BlockSpec((br, D), lambda i: (i, 0))


def _full_spec(shape):
    return pl.BlockSpec(shape, lambda i: tuple(0 for _ in shape))


def _sc_gather(grid_proj, mesh_proj, si3, di3):
    @functools.partial(
        pl.kernel,
        out_type=jax.ShapeDtypeStruct((E, D), f32),
        mesh=_sc_mesh,
        scratch_types=[
            pltpu.VMEM((NCH, CH), jnp.int32),
            pltpu.VMEM((NCH, CH), jnp.int32),
            pltpu.VMEM((CH, D), f32), pltpu.VMEM((CH, D), f32),
            pltpu.VMEM((CH, D), f32), pltpu.VMEM((CH, D), f32),
            pltpu.VMEM((CH, D), f32), pltpu.VMEM((CH, D), f32),
            pltpu.SemaphoreType.DMA, pltpu.SemaphoreType.DMA,
            pltpu.SemaphoreType.DMA, pltpu.SemaphoreType.DMA,
            pltpu.SemaphoreType.DMA, pltpu.SemaphoreType.DMA,
        ])
    def k(gp_hbm, mp_hbm, si_hbm, di_hbm, out_hbm,
          si_v, di_v, rs0, rd0, rs1, rd1, rs2, rd2, g0, g1, g2, w0, w1, w2):
        rs = (rs0, rs1, rs2)
        rd = (rd0, rd1, rd2)
        gsem = (g0, g1, g2)
        wsem = (w0, w1, w2)
        wid = lax.axis_index("s") * NC + lax.axis_index("c")
        base = wid * RW
        pltpu.sync_copy(si_hbm.at[wid], si_v)
        pltpu.sync_copy(di_hbm.at[wid], di_v)

        def gstart(b, c):
            pltpu.async_copy(gp_hbm.at[si_v.at[c]], rs[b], gsem[b])
            pltpu.async_copy(mp_hbm.at[di_v.at[c]], rd[b], gsem[b])

        def gwait(b):
            pltpu.make_async_copy(gp_hbm.at[pl.ds(0, CH)], rs[b], gsem[b]).wait()
            pltpu.make_async_copy(gp_hbm.at[pl.ds(0, CH)], rd[b], gsem[b]).wait()

        def add_rows(b):
            @pl.loop(0, CH)
            def _(r):
                for co in range(D // 16):
                    sl = pl.ds(co * 16, 16)
                    rs[b][r, sl] = rs[b][r, sl] + rd[b][r, sl]

        def wstart(b, c):
            pltpu.async_copy(rs[b], out_hbm.at[pl.ds(base + c * CH, CH)],
                             wsem[b])

        def wwait(b):
            pltpu.make_async_copy(rs[b], out_hbm.at[pl.ds(0, CH)],
                                  wsem[b]).wait()

        gstart(0, 0)

        @pl.loop(0, (NCH - 2) // 3)
        def _(p):
            c = 3 * p
            for h in range(3):
                bh, bn = h, (h + 1) % 3
                if h < 2:
                    @pl.when(p > 0)
                    def _():
                        wwait(bn)
                else:
                    wwait(bn)
                gwait(bh)
                gstart(bn, c + h + 1)
                add_rows(bh)
                wstart(bh, c + h)

        wwait(1)
        gwait(0)
        gstart(1, NCH - 1)
        add_rows(0)
        wstart(0, NCH - 2)
        gwait(1)
        add_rows(1)
        wstart(1, NCH - 1)
        wwait(2)
        wwait(0)
        wwait(1)

    return k(grid_proj, mesh_proj, si3, di3)


def _sc_scatter(e_out, di3, zeros_nd):
    @functools.partial(
        pl.kernel,
        out_type=jax.ShapeDtypeStruct((NC, N_DST, D), f32),
        mesh=_sc_mesh,
        scratch_types=[
            pltpu.VMEM((NCH, CH), jnp.int32),
            pltpu.VMEM((CH, D), f32), pltpu.VMEM((CH, D), f32),
            pltpu.VMEM_SHARED((N_DST, D), f32),
            pltpu.SemaphoreType.DMA, pltpu.SemaphoreType.DMA,
        ])
    def k(eo_hbm, di_hbm, z_hbm, out_hbm, di_v, ra, rb, agg_sh, sa, sb):
        rows = (ra, rb)
        sem = (sa, sb)
        cid = lax.axis_index("c")
        sid = lax.axis_index("s")
        wid = sid * NC + cid
        base = wid * RW
        pltpu.sync_copy(di_hbm.at[wid], di_v)

        @pl.when(sid == 0)
        def _():
            pltpu.sync_copy(z_hbm, agg_sh)
        plsc.subcore_barrier()

        def rstart(b, c):
            pltpu.async_copy(eo_hbm.at[pl.ds(base + c * CH, CH)], rows[b],
                             sem[b])

        def rwait(b):
            pltpu.make_async_copy(eo_hbm.at[pl.ds(0, CH)], rows[b],
                                  sem[b]).wait()

        def scat(b, c):
            pltpu.sync_copy(rows[b], agg_sh.at[di_v.at[c]], add=True)

        rstart(0, 0)

        @pl.loop(0, (NCH - 1) // 2)
        def _(p):
            c = 2 * p
            rwait(0)
            rstart(1, c + 1)
            scat(0, c)
            rwait(1)
            @pl.when(p < (NCH - 1) // 2 - 1)
            def _():
                rstart(0, c + 2)
            scat(1, c + 1)

        rstart(0, NCH - 1)
        rwait(0)
        scat(0, NCH - 1)

        plsc.subcore_barrier()

        @pl.when(sid == 0)
        def _():
            pltpu.sync_copy(agg_sh, out_hbm.at[cid])

    return k(e_out, di3, zeros_nd)


@jax.jit
def kernel(g2m_efeat, grid_nfeat, mesh_nfeat, edge_index,
           edge_W1, edge_b1, edge_W2, edge_b2, edge_g, edge_bt,
           src_W1, src_b1, src_W2, src_b2, src_g, src_bt,
           dst_W1, dst_b1, dst_W2, dst_b2, dst_g, dst_bt):
    si3 = edge_index[0].astype(jnp.int32).reshape(NW, NCH, CH)
    di3 = edge_index[1].astype(jnp.int32).reshape(NW, NCH, CH)

    w1e = edge_W1[:D]
    w1s = edge_W1[D:2 * D]
    w1d = edge_W1[2 * D:]
    dw1a = dst_W1[:D]
    dw1b = dst_W1[D:]

    def r2(b):
        return b.reshape(1, -1)

    BRG = 2000
    grid_proj = pl.pallas_call(
        _meshproj_body,
        grid=(N_SRC // BRG,),
        in_specs=[_row_spec(BRG), _full_spec((D, H))],
        out_specs=_row_spec(BRG),
        out_shape=jax.ShapeDtypeStruct((N_SRC, H), f32),
    )(grid_nfeat, w1s)

    BRM = 2000
    mesh_proj = pl.pallas_call(
        _meshproj_body,
        grid=(N_DST // BRM,),
        in_specs=[_row_spec(BRM), _full_spec((D, H))],
        out_specs=_row_spec(BRM),
        out_shape=jax.ShapeDtypeStruct((N_DST, H), f32),
    )(mesh_nfeat, w1d)

    gsum = _sc_gather(grid_proj, mesh_proj, si3, di3)

    grid_out = pl.pallas_call(
        _grid_body,
        grid=(N_SRC // BRG,),
        in_specs=[_row_spec(BRG), _full_spec((D, H)), _full_spec((1, H)),
                  _full_spec((H, D)), _full_spec((1, D)), _full_spec((1, D)),
                  _full_spec((1, D))],
        out_specs=_row_spec(BRG),
        out_shape=jax.ShapeDtypeStruct((N_SRC, D), f32),
    )(grid_nfeat, src_W1, r2(src_b1), src_W2, r2(src_b2), r2(src_g),
      r2(src_bt))

    BRE = 8000
    e_out = pl.pallas_call(
        _edge_body,
        grid=(E // BRE,),
        in_specs=[_row_spec(BRE), _row_spec(BRE),
                  _full_spec((D, H)), _full_spec((1, H)), _full_spec((H, D)),
                  _full_spec((1, D)), _full_spec((1, D)), _full_spec((1, D))],
        out_specs=_row_spec(BRE),
        out_shape=jax.ShapeDtypeStruct((E, D), f32),
    )(g2m_efeat, gsum, w1e, r2(edge_b1), edge_W2, r2(edge_b2),
      r2(edge_g), r2(edge_bt))

    parts = _sc_scatter(e_out, di3, jnp.zeros((N_DST, D), f32))

    BRD = 2000
    mesh_out = pl.pallas_call(
        _dst_body,
        grid=(N_DST // BRD,),
        in_specs=[_row_spec(BRD), _row_spec(BRD), _row_spec(BRD),
                  _full_spec((D, H)), _full_spec((D, H)), _full_spec((1, H)),
                  _full_spec((H, D)), _full_spec((1, D)), _full_spec((1, D)),
                  _full_spec((1, D))],
        out_specs=_row_spec(BRD),
        out_shape=jax.ShapeDtypeStruct((N_DST, D), f32),
    )(parts[0], parts[1], mesh_nfeat, dw1a, dw1b, r2(dst_b1), dst_W2,
      r2(dst_b2), r2(dst_g), r2(dst_bt))

    return (grid_out, mesh_out)


# speedup vs baseline: 5.2152x; 1.1630x over previous
import functools

import jax
import jax.numpy as jnp
from jax import lax
from jax.experimental import pallas as pl
from jax.experimental.pallas import tpu as pltpu
from jax.experimental.pallas import tpu_sc as plsc

N_SRC = 100000
N_DST = 10000
E = 320000
D = 128
H = 128

NC = 2
NS = 16
NW = NC * NS
RW = E // NW
CH = 80
NCH = RW // CH

f32 = jnp.float32

_sc_mesh = plsc.VectorSubcoreMesh(core_axis_name="c", subcore_axis_name="s")


def _ln(y, g, bt):
    mu = jnp.mean(y, axis=-1, keepdims=True)
    var = jnp.mean((y - mu) * (y - mu), axis=-1, keepdims=True)
    return (y - mu) * lax.rsqrt(var + 1e-5) * g + bt


def _silu(x):
    return x * lax.logistic(x)


def _grid_body(x_ref, wp_ref, w1_ref, b1_ref, w2_ref, b2_ref, g_ref, bt_ref,
               out_ref, proj_ref):
    x = x_ref[...]
    proj_ref[...] = jnp.dot(x, wp_ref[...], preferred_element_type=f32)
    h = _silu(jnp.dot(x, w1_ref[...], preferred_element_type=f32) + b1_ref[...])
    y = jnp.dot(h, w2_ref[...], preferred_element_type=f32) + b2_ref[...]
    out_ref[...] = x + _ln(y, g_ref[...], bt_ref[...])


def _meshproj_body(x_ref, wp_ref, proj_ref):
    proj_ref[...] = jnp.dot(x_ref[...], wp_ref[...], preferred_element_type=f32)


def _edge_body(ef_ref, gsum_ref, w1_ref, b1_ref, w2_ref, b2_ref, g_ref,
               bt_ref, out_ref):
    pre = (jnp.dot(ef_ref[...], w1_ref[...], preferred_element_type=f32)
           + gsum_ref[...] + b1_ref[...])
    h = _silu(pre)
    y = jnp.dot(h, w2_ref[...], preferred_element_type=f32) + b2_ref[...]
    out_ref[...] = _ln(y, g_ref[...], bt_ref[...])


def _dst_body(p0_ref, p1_ref, m_ref, w1a_ref, w1b_ref, b1_ref, w2_ref, b2_ref,
              g_ref, bt_ref, out_ref):
    agg = p0_ref[...] + p1_ref[...]
    m = m_ref[...]
    pre = (jnp.dot(agg, w1a_ref[...], preferred_element_type=f32)
           + jnp.dot(m, w1b_ref[...], preferred_element_type=f32)
           + b1_ref[...])
    h = _silu(pre)
    y = jnp.dot(h, w2_ref[...], preferred_element_type=f32) + b2_ref[...]
    out_ref[...] = m + _ln(y, g_ref[...], bt_ref[...])


def _row_spec(br):
    return pl.BlockSpec((br, D), lambda i: (i, 0))


def _full_spec(shape):
    return pl.BlockSpec(shape, lambda i: tuple(0 for _ in shape))


def _sc_gather(grid_proj, mesh_proj, si3, di3):
    @functools.partial(
        pl.kernel,
        out_type=jax.ShapeDtypeStruct((E, D), f32),
        mesh=_sc_mesh,
        scratch_types=[
            pltpu.VMEM((NCH, CH), jnp.int32),
            pltpu.VMEM((NCH, CH), jnp.int32),
            pltpu.VMEM((CH, D), f32), pltpu.VMEM((CH, D), f32),
            pltpu.VMEM((CH, D), f32), pltpu.VMEM((CH, D), f32),
            pltpu.VMEM((CH, D), f32), pltpu.VMEM((CH, D), f32),
            pltpu.VMEM((CH, D), f32), pltpu.VMEM((CH, D), f32),
            pltpu.SemaphoreType.DMA, pltpu.SemaphoreType.DMA,
            pltpu.SemaphoreType.DMA, pltpu.SemaphoreType.DMA,
            pltpu.SemaphoreType.DMA, pltpu.SemaphoreType.DMA,
            pltpu.SemaphoreType.DMA, pltpu.SemaphoreType.DMA,
        ])
    def k(gp_hbm, mp_hbm, si_hbm, di_hbm, out_hbm,
          si_v, di_v, rs0, rd0, rs1, rd1, rs2, rd2, rs3, rd3,
          g0, g1, g2, g3, w0, w1, w2, w3):
        rs = (rs0, rs1, rs2, rs3)
        rd = (rd0, rd1, rd2, rd3)
        gsem = (g0, g1, g2, g3)
        wsem = (w0, w1, w2, w3)
        wid = lax.axis_index("s") * NC + lax.axis_index("c")
        base = wid * RW
        pltpu.sync_copy(si_hbm.at[wid], si_v)
        pltpu.sync_copy(di_hbm.at[wid], di_v)

        def gstart(b, c):
            pltpu.async_copy(gp_hbm.at[si_v.at[c]], rs[b], gsem[b])
            pltpu.async_copy(mp_hbm.at[di_v.at[c]], rd[b], gsem[b])

        def gwait(b):
            pltpu.make_async_copy(gp_hbm.at[pl.ds(0, CH)], rs[b], gsem[b]).wait()
            pltpu.make_async_copy(gp_hbm.at[pl.ds(0, CH)], rd[b], gsem[b]).wait()

        def add_rows(b):
            @pl.loop(0, CH)
            def _(r):
                for co in range(D // 16):
                    sl = pl.ds(co * 16, 16)
                    rs[b][r, sl] = rs[b][r, sl] + rd[b][r, sl]

        def wstart(b, c):
            pltpu.async_copy(rs[b], out_hbm.at[pl.ds(base + c * CH, CH)],
                             wsem[b])

        def wwait(b):
            pltpu.make_async_copy(rs[b], out_hbm.at[pl.ds(0, CH)],
                                  wsem[b]).wait()

        gstart(0, 0)
        gstart(1, 1)

        @pl.loop(0, (NCH - 1) // 4)
        def _(p):
            c = 4 * p
            for h in range(4):
                bh, bn = h, (h + 2) % 4
                gwait(bh)
                if h < 2:
                    @pl.when(p > 0)
                    def _():
                        wwait(bn)
                        gstart(bn, c + h + 2)

                    @pl.when(p == 0)
                    def _():
                        gstart(bn, c + h + 2)
                elif h == 2:
                    wwait(bn)
                    gstart(bn, c + h + 2)
                else:
                    @pl.when(p < (NCH - 1) // 4 - 1)
                    def _():
                        wwait(bn)
                        gstart(bn, c + h + 2)
                add_rows(bh)
                wstart(bh, c + h)

        gwait(0)
        add_rows(0)
        wstart(0, NCH - 1)
        wwait(1)
        wwait(2)
        wwait(3)
        wwait(0)

    return k(grid_proj, mesh_proj, si3, di3)


def _sc_scatter(e_out, di3, zeros_nd):
    @functools.partial(
        pl.kernel,
        out_type=jax.ShapeDtypeStruct((NC, N_DST, D), f32),
        mesh=_sc_mesh,
        scratch_types=[
            pltpu.VMEM((NCH, CH), jnp.int32),
            pltpu.VMEM((CH, D), f32), pltpu.VMEM((CH, D), f32),
            pltpu.VMEM((CH, D), f32),
            pltpu.VMEM_SHARED((N_DST, D), f32),
            pltpu.SemaphoreType.DMA, pltpu.SemaphoreType.DMA,
            pltpu.SemaphoreType.DMA,
        ])
    def k(eo_hbm, di_hbm, z_hbm, out_hbm, di_v, ra, rb, rc, agg_sh,
          sa, sb, sc):
        rows = (ra, rb, rc)
        sem = (sa, sb, sc)
        cid = lax.axis_index("c")
        sid = lax.axis_index("s")
        wid = sid * NC + cid
        base = wid * RW
        pltpu.sync_copy(di_hbm.at[wid], di_v)

        @pl.when(sid == 0)
        def _():
            pltpu.sync_copy(z_hbm, agg_sh)
        plsc.subcore_barrier()

        def rstart(b, c):
            pltpu.async_copy(eo_hbm.at[pl.ds(base + c * CH, CH)], rows[b],
                             sem[b])

        def rwait(b):
            pltpu.make_async_copy(eo_hbm.at[pl.ds(0, CH)], rows[b],
                                  sem[b]).wait()

        def scat(b, c):
            pltpu.sync_copy(rows[b], agg_sh.at[di_v.at[c]], add=True)

        rstart(0, 0)
        rstart(1, 1)

        @pl.loop(0, (NCH - 2) // 3)
        def _(p):
            c = 3 * p
            for h in range(3):
                bh, bn = h, (h + 2) % 3
                rwait(bh)
                rstart(bn, c + h + 2)
                scat(bh, c + h)

        rwait(0)
        scat(0, NCH - 2)
        rwait(1)
        scat(1, NCH - 1)

        plsc.subcore_barrier()

        @pl.when(sid == 0)
        def _():
            pltpu.sync_copy(agg_sh, out_hbm.at[cid])

    return k(e_out, di3, zeros_nd)


@jax.jit
def kernel(g2m_efeat, grid_nfeat, mesh_nfeat, edge_index,
           edge_W1, edge_b1, edge_W2, edge_b2, edge_g, edge_bt,
           src_W1, src_b1, src_W2, src_b2, src_g, src_bt,
           dst_W1, dst_b1, dst_W2, dst_b2, dst_g, dst_bt):
    si3 = edge_index[0].astype(jnp.int32).reshape(NW, NCH, CH)
    di3 = edge_index[1].astype(jnp.int32).reshape(NW, NCH, CH)

    w1e = edge_W1[:D]
    w1s = edge_W1[D:2 * D]
    w1d = edge_W1[2 * D:]
    dw1a = dst_W1[:D]
    dw1b = dst_W1[D:]

    def r2(b):
        return b.reshape(1, -1)

    BRG = 2000
    grid_out, grid_proj = pl.pallas_call(
        _grid_body,
        grid=(N_SRC // BRG,),
        in_specs=[_row_spec(BRG), _full_spec((D, H)), _full_spec((D, H)),
                  _full_spec((1, H)), _full_spec((H, D)), _full_spec((1, D)),
                  _full_spec((1, D)), _full_spec((1, D))],
        out_specs=[_row_spec(BRG), _row_spec(BRG)],
        out_shape=[jax.ShapeDtypeStruct((N_SRC, D), f32),
                   jax.ShapeDtypeStruct((N_SRC, H), f32)],
    )(grid_nfeat, w1s, src_W1, r2(src_b1), src_W2, r2(src_b2), r2(src_g),
      r2(src_bt))

    BRM = 2000
    mesh_proj = pl.pallas_call(
        _meshproj_body,
        grid=(N_DST // BRM,),
        in_specs=[_row_spec(BRM), _full_spec((D, H))],
        out_specs=_row_spec(BRM),
        out_shape=jax.ShapeDtypeStruct((N_DST, H), f32),
    )(mesh_nfeat, w1d)

    gsum = _sc_gather(grid_proj, mesh_proj, si3, di3)

    BRE = 8000
    e_out = pl.pallas_call(
        _edge_body,
        grid=(E // BRE,),
        in_specs=[_row_spec(BRE), _row_spec(BRE),
                  _full_spec((D, H)), _full_spec((1, H)), _full_spec((H, D)),
                  _full_spec((1, D)), _full_spec((1, D)), _full_spec((1, D))],
        out_specs=_row_spec(BRE),
        out_shape=jax.ShapeDtypeStruct((E, D), f32),
    )(g2m_efeat, gsum, w1e, r2(edge_b1), edge_W2, r2(edge_b2),
      r2(edge_g), r2(edge_bt))

    parts = _sc_scatter(e_out, di3, jnp.zeros((N_DST, D), f32))

    BRD = 2000
    mesh_out = pl.pallas_call(
        _dst_body,
        grid=(N_DST // BRD,),
        in_specs=[_row_spec(BRD), _row_spec(BRD), _row_spec(BRD),
                  _full_spec((D, H)), _full_spec((D, H)), _full_spec((1, H)),
                  _full_spec((H, D)), _full_spec((1, D)), _full_spec((1, D)),
                  _full_spec((1, D))],
        out_specs=_row_spec(BRD),
        out_shape=jax.ShapeDtypeStruct((N_DST, D), f32),
    )(parts[0], parts[1], mesh_nfeat, dw1a, dw1b, r2(dst_b1), dst_W2,
      r2(dst_b2), r2(dst_g), r2(dst_bt))

    return (grid_out, mesh_out)
